# baseline (speedup 1.0000x reference)
import functools

import jax
import jax.numpy as jnp
from jax import lax
from jax.experimental import pallas as pl
from jax.experimental.pallas import tpu as pltpu
from jax.experimental.pallas import tpu_sc as plsc

HIDDEN = 64
SEQ = 200
NUM_WORKERS = 32
CHUNK = SEQ
GATHER_SPLIT = 2
SUB = CHUNK // GATHER_SPLIT


def _sc_body(idx_hbm, wtab_hbm, ptab_hbm, out_hbm, idx_v, rows_v, pos_v, sem):
    nc = 2
    wid = lax.axis_index("s") * nc + lax.axis_index("c")

    pltpu.sync_copy(ptab_hbm, pos_v)

    chunks_per_worker = idx_hbm.shape[0] // (NUM_WORKERS * GATHER_SPLIT)
    row0 = wid * chunks_per_worker * GATHER_SPLIT

    def chunk_body(g, carry):
        irow = row0 + g * GATHER_SPLIT
        pltpu.sync_copy(idx_hbm.at[pl.ds(irow, GATHER_SPLIT)], idx_v)
        cps = []
        for j in range(GATHER_SPLIT):
            cps.append(pltpu.async_copy(
                wtab_hbm.at[idx_v.at[j]],
                rows_v.at[pl.ds(j * SUB, SUB)],
                sem,
            ))
        for cp in cps:
            cp.wait()

        def row_body(r, carry2):
            for c in range(HIDDEN // 16):
                sl = pl.ds(c * 16, 16)
                v = rows_v[r, sl] + pos_v[r, sl]
                rows_v[r, sl] = jnp.maximum(v, 0.0)
            return carry2

        lax.fori_loop(0, CHUNK, row_body, 0, unroll=2)

        out_base = irow * SUB
        pltpu.sync_copy(rows_v, out_hbm.at[pl.ds(out_base, CHUNK)])
        return carry

    lax.fori_loop(0, chunks_per_worker, chunk_body, 0)


@functools.partial(jax.jit, static_argnames=())
def kernel(input_seq, word_table, pos_table):
    batch, seq = input_seq.shape
    total = batch * seq
    idx2d = input_seq.reshape(total // SUB, SUB).astype(jnp.int32)

    mesh = plsc.VectorSubcoreMesh(core_axis_name="c", subcore_axis_name="s")
    run = pl.kernel(
        _sc_body,
        out_type=jax.ShapeDtypeStruct((total, HIDDEN), jnp.float32),
        mesh=mesh,
        scratch_types=[
            pltpu.VMEM((GATHER_SPLIT, SUB), jnp.int32),
            pltpu.VMEM((CHUNK, HIDDEN), jnp.float32),
            pltpu.VMEM((SEQ, HIDDEN), jnp.float32),
            pltpu.SemaphoreType.DMA,
        ],
        compiler_params=pltpu.CompilerParams(use_tc_tiling_on_sc=False),
    )
    out = run(idx2d, word_table, pos_table)
    return out.reshape(batch, seq, HIDDEN)


# speedup vs baseline: 1.0814x; 1.0814x over previous
import functools

import jax
import jax.numpy as jnp
from jax import lax
from jax.experimental import pallas as pl
from jax.experimental.pallas import tpu as pltpu
from jax.experimental.pallas import tpu_sc as plsc

HIDDEN = 64
SEQ = 200
NUM_WORKERS = 32
CHUNK = SEQ
SUB = 100
BATCH_PER_W = 128
IDX_ROWS = BATCH_PER_W * 2


def _start_gather(wtab_hbm, idx_hbm, idx_b, buf, sem, row0, chunk):
    pltpu.sync_copy(idx_hbm.at[pl.ds(row0 + chunk * 2, 2)], idx_b)
    for j in range(2):
        pltpu.async_copy(
            wtab_hbm.at[idx_b.at[j]],
            buf.at[pl.ds(j * SUB, SUB)],
            sem,
        )


def _wait_gather(wtab_hbm, buf, sem):
    pltpu.make_async_copy(wtab_hbm.at[pl.ds(0, CHUNK)], buf, sem).wait()


def _combine(buf, pos_v):
    def row_body(r, carry):
        for c in range(HIDDEN // 16):
            sl = pl.ds(c * 16, 16)
            buf[r, sl] = jnp.maximum(buf[r, sl] + pos_v[r, sl], 0.0)
        return carry

    lax.fori_loop(0, CHUNK, row_body, 0, unroll=2)


def _sc_body(idx_hbm, wtab_hbm, ptab_hbm, out_hbm,
             idx_bufs, bufs, pos_v, sem0, sem1):
    nc = 2
    wid = lax.axis_index("s") * nc + lax.axis_index("c")
    batch0 = wid * BATCH_PER_W
    row0 = wid * IDX_ROWS

    pltpu.sync_copy(ptab_hbm, pos_v)

    buf0 = bufs.at[0]
    buf1 = bufs.at[1]
    idx0 = idx_bufs.at[0]
    idx1 = idx_bufs.at[1]

    _start_gather(wtab_hbm, idx_hbm, idx0, buf0, sem0, row0, 0)

    def pair_body(p, carry):
        g0 = 2 * p
        _start_gather(wtab_hbm, idx_hbm, idx1, buf1, sem1, row0, g0 + 1)
        _wait_gather(wtab_hbm, buf0, sem0)
        _combine(buf0, pos_v)
        pltpu.sync_copy(buf0, out_hbm.at[batch0 + g0])

        nxt = jnp.bitwise_and(g0 + 2, BATCH_PER_W - 1)
        _start_gather(wtab_hbm, idx_hbm, idx0, buf0, sem0, row0, nxt)
        _wait_gather(wtab_hbm, buf1, sem1)
        _combine(buf1, pos_v)
        pltpu.sync_copy(buf1, out_hbm.at[batch0 + g0 + 1])
        return carry

    lax.fori_loop(0, BATCH_PER_W // 2, pair_body, 0)
    _wait_gather(wtab_hbm, buf0, sem0)


@jax.jit
def kernel(input_seq, word_table, pos_table):
    batch, seq = input_seq.shape
    idx2d = input_seq.reshape(batch * seq // SUB, SUB).astype(jnp.int32)

    mesh = plsc.VectorSubcoreMesh(core_axis_name="c", subcore_axis_name="s")
    run = pl.kernel(
        _sc_body,
        out_type=jax.ShapeDtypeStruct((batch, seq, HIDDEN), jnp.float32),
        mesh=mesh,
        scratch_types=[
            pltpu.VMEM((2, 2, SUB), jnp.int32),
            pltpu.VMEM((2, CHUNK, HIDDEN), jnp.float32),
            pltpu.VMEM((SEQ, HIDDEN), jnp.float32),
            pltpu.SemaphoreType.DMA,
            pltpu.SemaphoreType.DMA,
        ],
        compiler_params=pltpu.CompilerParams(use_tc_tiling_on_sc=False),
    )
    return run(idx2d, word_table, pos_table)
